# baseline (speedup 1.0000x reference)
import functools

import jax
import jax.numpy as jnp
from jax import lax
from jax.experimental import pallas as pl
from jax.experimental.pallas import tpu as pltpu
from jax.experimental.pallas import tpu_sc as plsc

N = 10000
D = 128
E = 320000
NC = 2
NS = 16
NW = NC * NS
CHUNK = 128
EPW = 10240
NCHUNK = EPW // CHUNK
E_PAD = NW * EPW
N_PAD = 10240
ROWS_Z = N_PAD // NS

NBUF = 4
CH_D = 64
NCHUNK_D = EPW // CH_D
NQ_D = 4
NCH_Q = NCHUNK_D // NQ_D
NSUP_Q = NCH_Q // NBUF

CH_A = 128
NCHUNK_A = EPW // CH_A
NBUF_A = 2
NHALF = 2
NCH_H = NCHUNK_A // NHALF
NSUP_H = NCH_H // NBUF_A

_mesh = plsc.VectorSubcoreMesh(core_axis_name="c", subcore_axis_name="s")


@functools.partial(
    pl.kernel,
    out_type=jax.ShapeDtypeStruct((NC, N_PAD), jnp.float32),
    mesh=_mesh,
    scratch_types=[
        pltpu.VMEM_SHARED((N_PAD,), jnp.float32),
        pltpu.VMEM((ROWS_Z,), jnp.float32),
        pltpu.VMEM((CH_D,), jnp.float32),
        pltpu.VMEM((NCH_Q, CH_D), jnp.int32),
        pltpu.SemaphoreType.DMA((NBUF,)),
    ],
)
def _deg_kernel(dst_hbm, out_hbm, acc, zbuf, ones, didx, ssem):
    c = lax.axis_index("c")
    s = lax.axis_index("s")
    wid = c * NS + s

    @pl.loop(0, ROWS_Z // 16)
    def _(i):
        zbuf[pl.ds(i * 16, 16)] = jnp.zeros((16,), jnp.float32)

    @pl.loop(0, CH_D // 16)
    def _(i):
        ones[pl.ds(i * 16, 16)] = jnp.full((16,), 1.0, jnp.float32)

    pltpu.sync_copy(zbuf, acc.at[pl.ds(s * ROWS_Z, ROWS_Z)])
    plsc.subcore_barrier()

    @pl.loop(0, NQ_D)
    def _(q):
        qbase = pl.multiple_of(q * NCH_Q, 8)
        pltpu.sync_copy(dst_hbm.at[wid, pl.ds(qbase, NCH_Q)], didx)

        for b in range(NBUF):
            pltpu.async_copy(ones, acc.at[didx.at[b]], ssem.at[b], add=True)

        @pl.loop(0, NSUP_Q - 1)
        def _(jj):
            for b in range(NBUF):
                j = jj * NBUF + b
                pltpu.make_async_copy(ones, acc.at[didx.at[j]],
                                      ssem.at[b]).wait()
                pltpu.async_copy(ones, acc.at[didx.at[j + NBUF]], ssem.at[b],
                                 add=True)

        for b in range(NBUF):
            j = (NSUP_Q - 1) * NBUF + b
            pltpu.make_async_copy(ones, acc.at[didx.at[j]], ssem.at[b]).wait()

    plsc.subcore_barrier()
    pltpu.sync_copy(acc.at[pl.ds(s * ROWS_Z, ROWS_Z)],
                    out_hbm.at[c, pl.ds(s * ROWS_Z, ROWS_Z)])


@functools.partial(
    pl.kernel,
    out_type=jax.ShapeDtypeStruct((NC, N_PAD, D), jnp.float32),
    mesh=_mesh,
    scratch_types=[
        pltpu.VMEM_SHARED((N_PAD, D), jnp.float32),
        pltpu.VMEM((NBUF_A, CH_A, D), jnp.float32),
        pltpu.VMEM((NCH_H, CH_A), jnp.int32),
        pltpu.VMEM((NCH_H, CH_A), jnp.int32),
        pltpu.SemaphoreType.DMA((NBUF_A,)),
        pltpu.SemaphoreType.DMA((NBUF_A,)),
    ],
)
def _agg_kernel(y_hbm, src_hbm, dst_hbm, out_hbm, acc, rows, sidx, didx,
                gsem, ssem):
    c = lax.axis_index("c")
    s = lax.axis_index("s")
    wid = c * NS + s

    @pl.loop(0, CH_A)
    def _(r):
        @pl.loop(0, D // 16)
        def _(k):
            rows[pl.ds(0, 1), pl.ds(r, 1), pl.ds(k * 16, 16)] = (
                jnp.zeros((1, 1, 16), jnp.float32))

    @pl.loop(0, ROWS_Z // CH_A)
    def _(i):
        pltpu.sync_copy(rows.at[0],
                        acc.at[pl.ds(s * ROWS_Z + i * CH_A, CH_A)])

    plsc.subcore_barrier()

    @pl.loop(0, NHALF)
    def _(h):
        hbase = pl.multiple_of(h * NCH_H, 8)
        pltpu.sync_copy(src_hbm.at[wid, pl.ds(hbase, NCH_H)], sidx)
        pltpu.sync_copy(dst_hbm.at[wid, pl.ds(hbase, NCH_H)], didx)

        for b in range(NBUF_A):
            pltpu.async_copy(y_hbm.at[sidx.at[b]], rows.at[b], gsem.at[b])

        @pl.loop(0, NSUP_H)
        def _(jj):
            for b in range(NBUF_A):
                j = jj * NBUF_A + b
                pltpu.make_async_copy(y_hbm.at[sidx.at[j]], rows.at[b],
                                      gsem.at[b]).wait()
                pltpu.async_copy(rows.at[b], acc.at[didx.at[j]], ssem.at[b],
                                 add=True)
            for b in range(NBUF_A):
                j = jj * NBUF_A + b
                pltpu.make_async_copy(rows.at[b], acc.at[didx.at[j]],
                                      ssem.at[b]).wait()

                @pl.when(jj < NSUP_H - 1)
                def _():
                    pltpu.async_copy(y_hbm.at[sidx.at[j + NBUF_A]],
                                     rows.at[b], gsem.at[b])

    plsc.subcore_barrier()
    pltpu.sync_copy(acc.at[pl.ds(s * ROWS_Z, ROWS_Z)],
                    out_hbm.at[c, pl.ds(s * ROWS_Z, ROWS_Z)])


def _dense1_body(d0, d1, x_ref, w_ref, y_o, xw_o):
    dinv = lax.rsqrt(d0[...] + d1[...] + 1.0)
    xw = jnp.dot(x_ref[...], w_ref[...], preferred_element_type=jnp.float32)
    xw_o[...] = xw
    y_o[...] = dinv * xw


def _dense1(d0, d1, x, w):
    return pl.pallas_call(
        _dense1_body,
        out_shape=[jax.ShapeDtypeStruct((N, D), jnp.float32),
                   jax.ShapeDtypeStruct((N, D), jnp.float32)],
    )(d0, d1, x, w)


def _mid_body(d0, d1, a0, a1, xw, b, g, be, w, y_o, xw_o):
    dinv = lax.rsqrt(d0[...] + d1[...] + 1.0)
    h = dinv * (a0[...] + a1[...]) + (dinv * dinv) * xw[...] + b[...]
    mean = jnp.mean(h, axis=0, keepdims=True)
    var = jnp.mean((h - mean) ** 2, axis=0, keepdims=True)
    h = (h - mean) * lax.rsqrt(var + 1e-5) * g[...] + be[...]
    h = jnp.maximum(h, 0.0)
    xw2 = jnp.dot(h, w[...], preferred_element_type=jnp.float32)
    xw_o[...] = xw2
    y_o[...] = dinv * xw2


def _mid(d0, d1, a0, a1, xw, b, g, be, w):
    return pl.pallas_call(
        _mid_body,
        out_shape=[jax.ShapeDtypeStruct((N, D), jnp.float32),
                   jax.ShapeDtypeStruct((N, D), jnp.float32)],
    )(d0, d1, a0, a1, xw, b, g, be, w)


def _final_body(d0, d1, a0, a1, xw, b, o_ref):
    dinv = lax.rsqrt(d0[...] + d1[...] + 1.0)
    h = dinv * (a0[...] + a1[...]) + (dinv * dinv) * xw[...] + b[...]
    m = jnp.max(h, axis=1, keepdims=True)
    lse = jnp.log(jnp.sum(jnp.exp(h - m), axis=1, keepdims=True)) + m
    o_ref[...] = h - lse


def _final(d0, d1, a0, a1, xw, b):
    return pl.pallas_call(
        _final_body,
        out_shape=jax.ShapeDtypeStruct((N, D), jnp.float32),
    )(d0, d1, a0, a1, xw, b)


def kernel(x, edge_index, W1, b1, g1, be1, W2, b2, g2, be2, W3, b3):
    src = edge_index[0].astype(jnp.int32)
    dst = edge_index[1].astype(jnp.int32)
    pad_ids = jnp.arange(E_PAD - E, dtype=jnp.int32)
    src_f = jnp.concatenate([src, pad_ids % N])
    dst_f = jnp.concatenate([dst, N + pad_ids % (N_PAD - N)])
    src_p = src_f.reshape(NW, NCHUNK_A, CH_A)
    dst_p = dst_f.reshape(NW, NCHUNK_A, CH_A)
    dst_d = dst_f.reshape(NW, NCHUNK_D, CH_D)

    deg2 = _deg_kernel(dst_d)
    d0 = deg2[0, :N, None]
    d1 = deg2[1, :N, None]

    b1r, g1r, be1r = b1[None, :], g1[None, :], be1[None, :]
    b2r, g2r, be2r = b2[None, :], g2[None, :], be2[None, :]
    b3r = b3[None, :]

    y1, xw1 = _dense1(d0, d1, x, W1)
    a1 = _agg_kernel(y1, src_p, dst_p)
    y2, xw2 = _mid(d0, d1, a1[0, :N], a1[1, :N], xw1, b1r, g1r, be1r, W2)
    a2 = _agg_kernel(y2, src_p, dst_p)
    y3, xw3 = _mid(d0, d1, a2[0, :N], a2[1, :N], xw2, b2r, g2r, be2r, W3)
    a3 = _agg_kernel(y3, src_p, dst_p)
    return _final(d0, d1, a3[0, :N], a3[1, :N], xw3, b3r)


# speedup vs baseline: 1.1569x; 1.1569x over previous
import functools

import jax
import jax.numpy as jnp
from jax import lax
from jax.experimental import pallas as pl
from jax.experimental.pallas import tpu as pltpu
from jax.experimental.pallas import tpu_sc as plsc

N = 10000
D = 128
E = 320000
NC = 2
NS = 16
NW = NC * NS
CHUNK = 128
EPW = 10240
NCHUNK = EPW // CHUNK
E_PAD = NW * EPW
N_PAD = 10240
ROWS_Z = N_PAD // NS

NBUF = 4
CH_D = 64
NCHUNK_D = EPW // CH_D
NQ_D = 4
NCH_Q = NCHUNK_D // NQ_D
NSUP_Q = NCH_Q // NBUF

CH_A = 64
NCHUNK_A = EPW // CH_A
NBUF_A = 4
NHALF = 10
NCH_H = NCHUNK_A // NHALF
NSUP_H = NCH_H // NBUF_A

_mesh = plsc.VectorSubcoreMesh(core_axis_name="c", subcore_axis_name="s")


@functools.partial(
    pl.kernel,
    out_type=jax.ShapeDtypeStruct((NC, N_PAD), jnp.float32),
    mesh=_mesh,
    scratch_types=[
        pltpu.VMEM_SHARED((N_PAD,), jnp.float32),
        pltpu.VMEM((ROWS_Z,), jnp.float32),
        pltpu.VMEM((CH_D,), jnp.float32),
        pltpu.VMEM((NCH_Q, CH_D), jnp.int32),
        pltpu.SemaphoreType.DMA((NBUF,)),
    ],
)
def _deg_kernel(dst_hbm, out_hbm, acc, zbuf, ones, didx, ssem):
    c = lax.axis_index("c")
    s = lax.axis_index("s")
    wid = c * NS + s

    @pl.loop(0, ROWS_Z // 16)
    def _(i):
        zbuf[pl.ds(i * 16, 16)] = jnp.zeros((16,), jnp.float32)

    @pl.loop(0, CH_D // 16)
    def _(i):
        ones[pl.ds(i * 16, 16)] = jnp.full((16,), 1.0, jnp.float32)

    pltpu.sync_copy(zbuf, acc.at[pl.ds(s * ROWS_Z, ROWS_Z)])
    plsc.subcore_barrier()

    @pl.loop(0, NQ_D)
    def _(q):
        qbase = pl.multiple_of(q * NCH_Q, 8)
        pltpu.sync_copy(dst_hbm.at[wid, pl.ds(qbase, NCH_Q)], didx)

        for b in range(NBUF):
            pltpu.async_copy(ones, acc.at[didx.at[b]], ssem.at[b], add=True)

        @pl.loop(0, NSUP_Q - 1)
        def _(jj):
            for b in range(NBUF):
                j = jj * NBUF + b
                pltpu.make_async_copy(ones, acc.at[didx.at[j]],
                                      ssem.at[b]).wait()
                pltpu.async_copy(ones, acc.at[didx.at[j + NBUF]], ssem.at[b],
                                 add=True)

        for b in range(NBUF):
            j = (NSUP_Q - 1) * NBUF + b
            pltpu.make_async_copy(ones, acc.at[didx.at[j]], ssem.at[b]).wait()

    plsc.subcore_barrier()
    pltpu.sync_copy(acc.at[pl.ds(s * ROWS_Z, ROWS_Z)],
                    out_hbm.at[c, pl.ds(s * ROWS_Z, ROWS_Z)])


@functools.partial(
    pl.kernel,
    out_type=jax.ShapeDtypeStruct((NC, N_PAD, D), jnp.float32),
    mesh=_mesh,
    scratch_types=[
        pltpu.VMEM_SHARED((N_PAD, D), jnp.float32),
        pltpu.VMEM((NBUF_A, CH_A, D), jnp.float32),
        pltpu.VMEM((2, NCH_H, CH_A), jnp.int32),
        pltpu.VMEM((2, NCH_H, CH_A), jnp.int32),
        pltpu.SemaphoreType.DMA((NBUF_A,)),
        pltpu.SemaphoreType.DMA((NBUF_A,)),
        pltpu.SemaphoreType.DMA((2, 2)),
    ],
)
def _agg_kernel(y_hbm, src_hbm, dst_hbm, out_hbm, acc, rows, sidx, didx,
                gsem, ssem, isem):
    c = lax.axis_index("c")
    s = lax.axis_index("s")
    wid = c * NS + s

    @pl.loop(0, CH_A)
    def _(r):
        @pl.loop(0, D // 16)
        def _(k):
            rows[pl.ds(0, 1), pl.ds(r, 1), pl.ds(k * 16, 16)] = (
                jnp.zeros((1, 1, 16), jnp.float32))

    @pl.loop(0, ROWS_Z // CH_A)
    def _(i):
        pltpu.sync_copy(rows.at[0],
                        acc.at[pl.ds(s * ROWS_Z + i * CH_A, CH_A)])

    plsc.subcore_barrier()

    pltpu.async_copy(src_hbm.at[wid, pl.ds(0, NCH_H)], sidx.at[0],
                     isem.at[0, 0])
    pltpu.async_copy(dst_hbm.at[wid, pl.ds(0, NCH_H)], didx.at[0],
                     isem.at[0, 1])

    @pl.loop(0, NHALF)
    def _(h):
        slot = lax.rem(h, 2)
        nxt = 1 - slot
        hbase = pl.multiple_of(h * NCH_H, 8)
        pltpu.make_async_copy(src_hbm.at[wid, pl.ds(hbase, NCH_H)],
                              sidx.at[slot], isem.at[slot, 0]).wait()
        pltpu.make_async_copy(dst_hbm.at[wid, pl.ds(hbase, NCH_H)],
                              didx.at[slot], isem.at[slot, 1]).wait()

        @pl.when(h < NHALF - 1)
        def _():
            nbase = pl.multiple_of((h + 1) * NCH_H, 8)
            pltpu.async_copy(src_hbm.at[wid, pl.ds(nbase, NCH_H)],
                             sidx.at[nxt], isem.at[nxt, 0])
            pltpu.async_copy(dst_hbm.at[wid, pl.ds(nbase, NCH_H)],
                             didx.at[nxt], isem.at[nxt, 1])

        for b in range(NBUF_A):
            pltpu.async_copy(y_hbm.at[sidx.at[slot, b]], rows.at[b],
                             gsem.at[b])

        @pl.loop(0, NSUP_H)
        def _(jj):
            for b in range(NBUF_A):
                j = jj * NBUF_A + b
                pltpu.make_async_copy(y_hbm.at[sidx.at[slot, j]], rows.at[b],
                                      gsem.at[b]).wait()
                pltpu.async_copy(rows.at[b], acc.at[didx.at[slot, j]],
                                 ssem.at[b], add=True)
            for b in range(NBUF_A):
                j = jj * NBUF_A + b
                pltpu.make_async_copy(rows.at[b], acc.at[didx.at[slot, j]],
                                      ssem.at[b]).wait()

                @pl.when(jj < NSUP_H - 1)
                def _():
                    pltpu.async_copy(y_hbm.at[sidx.at[slot, j + NBUF_A]],
                                     rows.at[b], gsem.at[b])

    plsc.subcore_barrier()
    pltpu.sync_copy(acc.at[pl.ds(s * ROWS_Z, ROWS_Z)],
                    out_hbm.at[c, pl.ds(s * ROWS_Z, ROWS_Z)])


def _dense1_body(d0, d1, x_ref, w_ref, y_o, xw_o):
    dinv = lax.rsqrt(d0[...] + d1[...] + 1.0)
    xw = jnp.dot(x_ref[...], w_ref[...], preferred_element_type=jnp.float32)
    xw_o[...] = xw
    y_o[...] = dinv * xw


def _dense1(d0, d1, x, w):
    return pl.pallas_call(
        _dense1_body,
        out_shape=[jax.ShapeDtypeStruct((N, D), jnp.float32),
                   jax.ShapeDtypeStruct((N, D), jnp.float32)],
    )(d0, d1, x, w)


def _mid_body(d0, d1, a0, a1, xw, b, g, be, w, y_o, xw_o):
    dinv = lax.rsqrt(d0[...] + d1[...] + 1.0)
    h = dinv * (a0[...] + a1[...]) + (dinv * dinv) * xw[...] + b[...]
    mean = jnp.mean(h, axis=0, keepdims=True)
    var = jnp.mean((h - mean) ** 2, axis=0, keepdims=True)
    h = (h - mean) * lax.rsqrt(var + 1e-5) * g[...] + be[...]
    h = jnp.maximum(h, 0.0)
    xw2 = jnp.dot(h, w[...], preferred_element_type=jnp.float32)
    xw_o[...] = xw2
    y_o[...] = dinv * xw2


def _mid(d0, d1, a0, a1, xw, b, g, be, w):
    return pl.pallas_call(
        _mid_body,
        out_shape=[jax.ShapeDtypeStruct((N, D), jnp.float32),
                   jax.ShapeDtypeStruct((N, D), jnp.float32)],
    )(d0, d1, a0, a1, xw, b, g, be, w)


def _final_body(d0, d1, a0, a1, xw, b, o_ref):
    dinv = lax.rsqrt(d0[...] + d1[...] + 1.0)
    h = dinv * (a0[...] + a1[...]) + (dinv * dinv) * xw[...] + b[...]
    m = jnp.max(h, axis=1, keepdims=True)
    lse = jnp.log(jnp.sum(jnp.exp(h - m), axis=1, keepdims=True)) + m
    o_ref[...] = h - lse


def _final(d0, d1, a0, a1, xw, b):
    return pl.pallas_call(
        _final_body,
        out_shape=jax.ShapeDtypeStruct((N, D), jnp.float32),
    )(d0, d1, a0, a1, xw, b)


def kernel(x, edge_index, W1, b1, g1, be1, W2, b2, g2, be2, W3, b3):
    src = edge_index[0].astype(jnp.int32)
    dst = edge_index[1].astype(jnp.int32)
    pad_ids = jnp.arange(E_PAD - E, dtype=jnp.int32)
    src_f = jnp.concatenate([src, pad_ids % N])
    dst_f = jnp.concatenate([dst, N + pad_ids % (N_PAD - N)])
    src_p = src_f.reshape(NW, NCHUNK_A, CH_A)
    dst_p = dst_f.reshape(NW, NCHUNK_A, CH_A)
    dst_d = dst_f.reshape(NW, NCHUNK_D, CH_D)

    deg2 = _deg_kernel(dst_d)
    d0 = deg2[0, :N, None]
    d1 = deg2[1, :N, None]

    b1r, g1r, be1r = b1[None, :], g1[None, :], be1[None, :]
    b2r, g2r, be2r = b2[None, :], g2[None, :], be2[None, :]
    b3r = b3[None, :]

    y1, xw1 = _dense1(d0, d1, x, W1)
    a1 = _agg_kernel(y1, src_p, dst_p)
    y2, xw2 = _mid(d0, d1, a1[0, :N], a1[1, :N], xw1, b1r, g1r, be1r, W2)
    a2 = _agg_kernel(y2, src_p, dst_p)
    y3, xw3 = _mid(d0, d1, a2[0, :N], a2[1, :N], xw2, b2r, g2r, be2r, W3)
    a3 = _agg_kernel(y3, src_p, dst_p)
    return _final(d0, d1, a3[0, :N], a3[1, :N], xw3, b3r)


# speedup vs baseline: 1.2061x; 1.0425x over previous
import functools

import jax
import jax.numpy as jnp
from jax import lax
from jax.experimental import pallas as pl
from jax.experimental.pallas import tpu as pltpu
from jax.experimental.pallas import tpu_sc as plsc

N = 10000
D = 128
E = 320000
NC = 2
NS = 16
NW = NC * NS
CHUNK = 128
EPW = 10240
NCHUNK = EPW // CHUNK
E_PAD = NW * EPW
N_PAD = 10240
ROWS_Z = N_PAD // NS

NBUF = 4
CH_D = 64
NCHUNK_D = EPW // CH_D
NQ_D = 4
NCH_Q = NCHUNK_D // NQ_D
NSUP_Q = NCH_Q // NBUF

CH_A = 64
NCHUNK_A = EPW // CH_A
NBUF_A = 4
NHALF = 10
NCH_H = NCHUNK_A // NHALF
NSUP_H = NCH_H // NBUF_A

_mesh = plsc.VectorSubcoreMesh(core_axis_name="c", subcore_axis_name="s")


@functools.partial(
    pl.kernel,
    out_type=jax.ShapeDtypeStruct((NC, N_PAD), jnp.float32),
    mesh=_mesh,
    scratch_types=[
        pltpu.VMEM_SHARED((N_PAD,), jnp.float32),
        pltpu.VMEM((ROWS_Z,), jnp.float32),
        pltpu.VMEM((CH_D,), jnp.float32),
        pltpu.VMEM((NCH_Q, CH_D), jnp.int32),
        pltpu.SemaphoreType.DMA((NBUF,)),
    ],
)
def _deg_kernel(dst_hbm, out_hbm, acc, zbuf, ones, didx, ssem):
    c = lax.axis_index("c")
    s = lax.axis_index("s")
    wid = c * NS + s

    @pl.loop(0, ROWS_Z // 16)
    def _(i):
        zbuf[pl.ds(i * 16, 16)] = jnp.zeros((16,), jnp.float32)

    @pl.loop(0, CH_D // 16)
    def _(i):
        ones[pl.ds(i * 16, 16)] = jnp.full((16,), 1.0, jnp.float32)

    pltpu.sync_copy(zbuf, acc.at[pl.ds(s * ROWS_Z, ROWS_Z)])
    plsc.subcore_barrier()

    @pl.loop(0, NQ_D)
    def _(q):
        qbase = pl.multiple_of(q * NCH_Q, 8)
        pltpu.sync_copy(dst_hbm.at[wid, pl.ds(qbase, NCH_Q)], didx)

        for b in range(NBUF):
            pltpu.async_copy(ones, acc.at[didx.at[b]], ssem.at[b], add=True)

        @pl.loop(0, NSUP_Q - 1)
        def _(jj):
            for b in range(NBUF):
                j = jj * NBUF + b
                pltpu.make_async_copy(ones, acc.at[didx.at[j]],
                                      ssem.at[b]).wait()
                pltpu.async_copy(ones, acc.at[didx.at[j + NBUF]], ssem.at[b],
                                 add=True)

        for b in range(NBUF):
            j = (NSUP_Q - 1) * NBUF + b
            pltpu.make_async_copy(ones, acc.at[didx.at[j]], ssem.at[b]).wait()

    plsc.subcore_barrier()
    pltpu.sync_copy(acc.at[pl.ds(s * ROWS_Z, ROWS_Z)],
                    out_hbm.at[c, pl.ds(s * ROWS_Z, ROWS_Z)])


@functools.partial(
    pl.kernel,
    out_type=jax.ShapeDtypeStruct((NC, N_PAD, D), jnp.float32),
    mesh=_mesh,
    scratch_types=[
        pltpu.VMEM_SHARED((N_PAD, D), jnp.float32),
        pltpu.VMEM((NBUF_A, CH_A, D), jnp.float32),
        pltpu.VMEM((2, NCH_H, CH_A), jnp.int32),
        pltpu.VMEM((2, NCH_H, CH_A), jnp.int32),
        pltpu.SemaphoreType.DMA((NBUF_A,)),
        pltpu.SemaphoreType.DMA((NBUF_A,)),
        pltpu.SemaphoreType.DMA((2, 2)),
    ],
)
def _agg_kernel(y_hbm, src_hbm, dst_hbm, out_hbm, acc, rows, sidx, didx,
                gsem, ssem, isem):
    c = lax.axis_index("c")
    s = lax.axis_index("s")
    wid = c * NS + s

    @pl.loop(0, CH_A)
    def _(r):
        @pl.loop(0, D // 16)
        def _(k):
            rows[pl.ds(0, 1), pl.ds(r, 1), pl.ds(k * 16, 16)] = (
                jnp.zeros((1, 1, 16), jnp.float32))

    @pl.loop(0, ROWS_Z // CH_A)
    def _(i):
        pltpu.sync_copy(rows.at[0],
                        acc.at[pl.ds(s * ROWS_Z + i * CH_A, CH_A)])

    plsc.subcore_barrier()

    pltpu.async_copy(src_hbm.at[wid, pl.ds(0, NCH_H)], sidx.at[0],
                     isem.at[0, 0])
    pltpu.async_copy(dst_hbm.at[wid, pl.ds(0, NCH_H)], didx.at[0],
                     isem.at[0, 1])
    pltpu.make_async_copy(src_hbm.at[wid, pl.ds(0, NCH_H)], sidx.at[0],
                          isem.at[0, 0]).wait()
    pltpu.make_async_copy(dst_hbm.at[wid, pl.ds(0, NCH_H)], didx.at[0],
                          isem.at[0, 1]).wait()
    for b in range(NBUF_A):
        pltpu.async_copy(y_hbm.at[sidx.at[0, b]], rows.at[b], gsem.at[b])

    @pl.loop(0, NHALF)
    def _(h):
        slot = lax.rem(h, 2)
        nxt = 1 - slot

        @pl.when(h < NHALF - 1)
        def _():
            nbase = pl.multiple_of((h + 1) * NCH_H, 8)
            pltpu.async_copy(src_hbm.at[wid, pl.ds(nbase, NCH_H)],
                             sidx.at[nxt], isem.at[nxt, 0])
            pltpu.async_copy(dst_hbm.at[wid, pl.ds(nbase, NCH_H)],
                             didx.at[nxt], isem.at[nxt, 1])

        @pl.loop(0, NSUP_H)
        def _(jj):
            for b in range(NBUF_A):
                j = jj * NBUF_A + b
                pltpu.make_async_copy(y_hbm.at[sidx.at[slot, j]], rows.at[b],
                                      gsem.at[b]).wait()
                pltpu.async_copy(rows.at[b], acc.at[didx.at[slot, j]],
                                 ssem.at[b], add=True)

            @pl.when((jj == NSUP_H - 1) & (h < NHALF - 1))
            def _():
                hbase = pl.multiple_of((h + 1) * NCH_H, 8)
                pltpu.make_async_copy(src_hbm.at[wid, pl.ds(hbase, NCH_H)],
                                      sidx.at[nxt], isem.at[nxt, 0]).wait()
                pltpu.make_async_copy(dst_hbm.at[wid, pl.ds(hbase, NCH_H)],
                                      didx.at[nxt], isem.at[nxt, 1]).wait()

            for b in range(NBUF_A):
                j = jj * NBUF_A + b
                pltpu.make_async_copy(rows.at[b], acc.at[didx.at[slot, j]],
                                      ssem.at[b]).wait()

                @pl.when(jj < NSUP_H - 1)
                def _():
                    pltpu.async_copy(y_hbm.at[sidx.at[slot, j + NBUF_A]],
                                     rows.at[b], gsem.at[b])

                @pl.when((jj == NSUP_H - 1) & (h < NHALF - 1))
                def _():
                    pltpu.async_copy(y_hbm.at[sidx.at[nxt, b]],
                                     rows.at[b], gsem.at[b])

    plsc.subcore_barrier()
    pltpu.sync_copy(acc.at[pl.ds(s * ROWS_Z, ROWS_Z)],
                    out_hbm.at[c, pl.ds(s * ROWS_Z, ROWS_Z)])


def _dense1_body(d0, d1, x_ref, w_ref, y_o, xw_o):
    dinv = lax.rsqrt(d0[...] + d1[...] + 1.0)
    xw = jnp.dot(x_ref[...], w_ref[...], preferred_element_type=jnp.float32)
    xw_o[...] = xw
    y_o[...] = dinv * xw


def _dense1(d0, d1, x, w):
    return pl.pallas_call(
        _dense1_body,
        out_shape=[jax.ShapeDtypeStruct((N, D), jnp.float32),
                   jax.ShapeDtypeStruct((N, D), jnp.float32)],
    )(d0, d1, x, w)


def _mid_body(d0, d1, a0, a1, xw, b, g, be, w, y_o, xw_o):
    dinv = lax.rsqrt(d0[...] + d1[...] + 1.0)
    h = dinv * (a0[...] + a1[...]) + (dinv * dinv) * xw[...] + b[...]
    mean = jnp.mean(h, axis=0, keepdims=True)
    var = jnp.mean((h - mean) ** 2, axis=0, keepdims=True)
    h = (h - mean) * lax.rsqrt(var + 1e-5) * g[...] + be[...]
    h = jnp.maximum(h, 0.0)
    xw2 = jnp.dot(h, w[...], preferred_element_type=jnp.float32)
    xw_o[...] = xw2
    y_o[...] = dinv * xw2


def _mid(d0, d1, a0, a1, xw, b, g, be, w):
    return pl.pallas_call(
        _mid_body,
        out_shape=[jax.ShapeDtypeStruct((N, D), jnp.float32),
                   jax.ShapeDtypeStruct((N, D), jnp.float32)],
    )(d0, d1, a0, a1, xw, b, g, be, w)


def _final_body(d0, d1, a0, a1, xw, b, o_ref):
    dinv = lax.rsqrt(d0[...] + d1[...] + 1.0)
    h = dinv * (a0[...] + a1[...]) + (dinv * dinv) * xw[...] + b[...]
    m = jnp.max(h, axis=1, keepdims=True)
    lse = jnp.log(jnp.sum(jnp.exp(h - m), axis=1, keepdims=True)) + m
    o_ref[...] = h - lse


def _final(d0, d1, a0, a1, xw, b):
    return pl.pallas_call(
        _final_body,
        out_shape=jax.ShapeDtypeStruct((N, D), jnp.float32),
    )(d0, d1, a0, a1, xw, b)


def kernel(x, edge_index, W1, b1, g1, be1, W2, b2, g2, be2, W3, b3):
    src = edge_index[0].astype(jnp.int32)
    dst = edge_index[1].astype(jnp.int32)
    pad_ids = jnp.arange(E_PAD - E, dtype=jnp.int32)
    src_f = jnp.concatenate([src, pad_ids % N])
    dst_f = jnp.concatenate([dst, N + pad_ids % (N_PAD - N)])
    src_p = src_f.reshape(NW, NCHUNK_A, CH_A)
    dst_p = dst_f.reshape(NW, NCHUNK_A, CH_A)
    dst_d = dst_f.reshape(NW, NCHUNK_D, CH_D)

    deg2 = _deg_kernel(dst_d)
    d0 = deg2[0, :N, None]
    d1 = deg2[1, :N, None]

    b1r, g1r, be1r = b1[None, :], g1[None, :], be1[None, :]
    b2r, g2r, be2r = b2[None, :], g2[None, :], be2[None, :]
    b3r = b3[None, :]

    y1, xw1 = _dense1(d0, d1, x, W1)
    a1 = _agg_kernel(y1, src_p, dst_p)
    y2, xw2 = _mid(d0, d1, a1[0, :N], a1[1, :N], xw1, b1r, g1r, be1r, W2)
    a2 = _agg_kernel(y2, src_p, dst_p)
    y3, xw3 = _mid(d0, d1, a2[0, :N], a2[1, :N], xw2, b2r, g2r, be2r, W3)
    a3 = _agg_kernel(y3, src_p, dst_p)
    return _final(d0, d1, a3[0, :N], a3[1, :N], xw3, b3r)
